# baseline (speedup 1.0000x reference)
import functools

import jax
import jax.numpy as jnp
from jax import lax
from jax.experimental import pallas as pl
from jax.experimental.pallas import tpu as pltpu
from jax.experimental.pallas import tpu_sc as plsc

NC = 2
NS = 16
NW = NC * NS

CHUNK = 128


@functools.lru_cache(maxsize=None)
def _build(batch, dim):
    b_per_w = batch // NW
    n_chunks = b_per_w // CHUNK
    mesh = plsc.VectorSubcoreMesh(core_axis_name="c", subcore_axis_name="s")

    @functools.partial(
        pl.kernel,
        mesh=mesh,
        out_type=jax.ShapeDtypeStruct((batch, dim), jnp.float32),
        scratch_types=[
            pltpu.VMEM((n_chunks, CHUNK), jnp.int32),
            pltpu.VMEM((b_per_w, dim), jnp.float32),
            pltpu.SemaphoreType.DMA,
        ],
        compiler_params=pltpu.CompilerParams(use_tc_tiling_on_sc=False),
    )
    def gather_kernel(idx_hbm, table_hbm, out_hbm, idx_v, rows_v, sem):
        wid = lax.axis_index("s") * NC + lax.axis_index("c")
        pltpu.sync_copy(idx_hbm.at[wid], idx_v)
        copies = []
        for j in range(n_chunks):
            copies.append(
                pltpu.async_copy(
                    table_hbm.at[idx_v.at[j]],
                    rows_v.at[pl.ds(j * CHUNK, CHUNK)],
                    sem,
                )
            )
        for c in copies:
            c.wait()
        pltpu.sync_copy(rows_v, out_hbm.at[pl.ds(wid * b_per_w, b_per_w)])

    return gather_kernel


def kernel(y, table):
    batch = y.shape[0]
    dim = table.shape[1]
    b_per_w = batch // NW
    idx = y.reshape(NW, b_per_w // CHUNK, CHUNK).astype(jnp.int32)
    out = _build(batch, dim)(idx, table)
    return out.reshape(batch, 1, dim)


# speedup vs baseline: 1.6681x; 1.6681x over previous
import functools

import jax
import jax.numpy as jnp
from jax import lax
from jax.experimental import pallas as pl
from jax.experimental.pallas import tpu as pltpu
from jax.experimental.pallas import tpu_sc as plsc

NC = 2
NS = 16
NW = NC * NS

CHUNK = 64
LANES = 16


@functools.lru_cache(maxsize=None)
def _build(batch, dim):
    b_per_w = batch // NW
    n_chunks = b_per_w // CHUNK
    mesh = plsc.VectorSubcoreMesh(core_axis_name="c", subcore_axis_name="s")

    @functools.partial(
        pl.kernel,
        mesh=mesh,
        out_type=jax.ShapeDtypeStruct((batch * dim,), jnp.float32),
        scratch_types=[
            pltpu.VMEM((b_per_w,), jnp.int32),
            pltpu.VMEM((2, CHUNK * dim), jnp.float32),
            pltpu.SemaphoreType.DMA,
            pltpu.SemaphoreType.DMA,
            pltpu.SemaphoreType.DMA,
            pltpu.SemaphoreType.DMA,
        ],
    )
    def gather_kernel(idx_hbm, table_hbm, out_hbm,
                      idx_v, buf_v, sg0, sg1, so0, so1):
        wid = lax.axis_index("s") * NC + lax.axis_index("c")
        base = wid * b_per_w
        pltpu.sync_copy(idx_hbm.at[pl.ds(base, b_per_w)], idx_v)

        sg = (sg0, sg1)
        so = (so0, so1)

        def start_chunk(j):
            copies = []
            b = j % 2
            for i in range(CHUNK):
                k = j * CHUNK + i
                if i % LANES == 0:
                    v16 = idx_v[pl.ds(k, LANES)]
                t = v16[i % LANES]
                copies.append(pltpu.async_copy(
                    table_hbm.at[t],
                    buf_v.at[b, pl.ds(i * dim, dim)],
                    sg[b],
                ))
            return copies

        gathers = [start_chunk(0), start_chunk(1)]
        for j in range(n_chunks):
            b = j % 2
            for c in gathers[b]:
                c.wait()
            pltpu.sync_copy(
                buf_v.at[b],
                out_hbm.at[pl.ds((base + j * CHUNK) * dim, CHUNK * dim)],
            )
            if j + 2 < n_chunks:
                gathers[b] = start_chunk(j + 2)

    return gather_kernel


def kernel(y, table):
    batch = y.shape[0]
    dim = table.shape[1]
    idx = y.reshape(batch).astype(jnp.int32)
    out = _build(batch, dim)(idx, table)
    return out.reshape(batch, 1, dim)


# speedup vs baseline: 2.6661x; 1.5982x over previous
import functools

import jax
import jax.numpy as jnp
from jax import lax
from jax.experimental import pallas as pl
from jax.experimental.pallas import tpu as pltpu
from jax.experimental.pallas import tpu_sc as plsc

NC = 2
NS = 16
NW = NC * NS

LANES = 16
BLK = 128
DEPTH = 4


@functools.lru_cache(maxsize=None)
def _build(batch, dim):
    b_per_w = batch // NW
    n_grp = b_per_w // LANES
    mesh = plsc.VectorSubcoreMesh(core_axis_name="c", subcore_axis_name="s")

    @functools.partial(
        pl.kernel,
        mesh=mesh,
        out_type=jax.ShapeDtypeStruct((batch, dim), jnp.float32),
        scratch_types=[
            pltpu.VMEM((b_per_w,), jnp.int32),
            pltpu.VMEM((DEPTH, dim, BLK), jnp.float32),
            pltpu.VMEM((b_per_w, dim), jnp.float32),
            pltpu.SemaphoreType.DMA((DEPTH,)),
        ],
    )
    def gather_kernel(idx_hbm, tab_hbm, out_hbm, idx_v, g_v, o_v, sems):
        wid = lax.axis_index("s") * NC + lax.axis_index("c")
        base = pl.multiple_of(wid * b_per_w, BLK)
        pltpu.sync_copy(idx_hbm.at[pl.ds(base, b_per_w)], idx_v)

        iota = lax.iota(jnp.int32, LANES)
        onehot = [
            jnp.maximum(1 - jnp.abs(iota - l), 0).astype(jnp.float32)
            for l in range(LANES)
        ]

        def fetch(t, slot):
            t0 = pl.multiple_of((t >> 7) << 7, BLK)
            return pltpu.async_copy(
                tab_hbm.at[:, pl.ds(t0, BLK)],
                g_v.at[slot],
                sems.at[slot],
            )

        v16p = idx_v[pl.ds(0, LANES)]
        for k in range(DEPTH):
            fetch(v16p[k], k)

        def group_body(g):
            gbase = pl.multiple_of(g * LANES, LANES)
            v16 = idx_v[pl.ds(gbase, LANES)]
            vn16 = idx_v[pl.ds(pl.multiple_of(
                jnp.minimum(g + 1, n_grp - 1) * LANES, LANES), LANES)]
            for k in range(LANES):
                s = k % DEPTH
                i = gbase + k
                pltpu.make_async_copy(
                    tab_hbm.at[:, pl.ds(0, BLK)], g_v.at[s], sems.at[s]
                ).wait()
                t = v16[k]
                rl = t & 127
                for q in range(dim // LANES):
                    acc = jnp.zeros((LANES,), jnp.float32)
                    for l in range(LANES):
                        d = q * LANES + l
                        w = g_v[s, d, pl.ds(rl, LANES)]
                        acc = acc + onehot[l] * jnp.full(
                            (LANES,), w[0], jnp.float32)
                    o_v[i, pl.ds(q * LANES, LANES)] = acc
                tn = v16[k + DEPTH] if k + DEPTH < LANES else vn16[k - 12]

                @pl.when(i + DEPTH < b_per_w)
                def _():
                    fetch(tn, s)

        pl.loop(0, n_grp)(group_body)
        pltpu.sync_copy(o_v, out_hbm.at[pl.ds(base, b_per_w)])

    return gather_kernel


def kernel(y, table):
    batch = y.shape[0]
    dim = table.shape[1]
    idx = y.reshape(batch).astype(jnp.int32)
    out = _build(batch, dim)(idx, table.T)
    return out.reshape(batch, 1, dim)


# speedup vs baseline: 2.9798x; 1.1177x over previous
import functools

import jax
import jax.numpy as jnp
from jax import lax
from jax.experimental import pallas as pl
from jax.experimental.pallas import tpu as pltpu
from jax.experimental.pallas import tpu_sc as plsc

NC = 2
NS = 16
NW = NC * NS

LANES = 16
BLK = 128
DEPTH = 8


@functools.lru_cache(maxsize=None)
def _build(batch, dim):
    b_per_w = batch // NW
    n_grp = b_per_w // LANES
    mesh = plsc.VectorSubcoreMesh(core_axis_name="c", subcore_axis_name="s")

    @functools.partial(
        pl.kernel,
        mesh=mesh,
        out_type=jax.ShapeDtypeStruct((batch, dim), jnp.float32),
        scratch_types=[
            pltpu.VMEM((b_per_w,), jnp.int32),
            pltpu.VMEM((DEPTH, dim, BLK), jnp.float32),
            pltpu.VMEM((b_per_w // 2, dim), jnp.float32),
            pltpu.SemaphoreType.DMA((DEPTH,)),
        ],
    )
    def gather_kernel(idx_hbm, tab_hbm, out_hbm, idx_v, g_v, o_v, sems):
        wid = lax.axis_index("s") * NC + lax.axis_index("c")
        base = pl.multiple_of(wid * b_per_w, BLK)
        pltpu.sync_copy(idx_hbm.at[pl.ds(base, b_per_w)], idx_v)

        iota = lax.iota(jnp.int32, LANES)
        onehot = [
            jnp.maximum(1 - jnp.abs(iota - l), 0).astype(jnp.float32)
            for l in range(LANES)
        ]

        def fetch(t, slot):
            t0 = pl.multiple_of((t >> 7) << 7, BLK)
            return pltpu.async_copy(
                tab_hbm.at[:, pl.ds(t0, BLK)],
                g_v.at[slot],
                sems.at[slot],
            )

        v16p = idx_v[pl.ds(0, LANES)]
        for k in range(DEPTH):
            fetch(v16p[k], k)

        def group_body(g):
            gbase = pl.multiple_of(g * LANES, LANES)
            v16 = idx_v[pl.ds(gbase, LANES)]
            vn16 = idx_v[pl.ds(pl.multiple_of(
                jnp.minimum(g + 1, n_grp - 1) * LANES, LANES), LANES)]
            for k in range(LANES):
                s = k % DEPTH
                i = gbase + k
                pltpu.make_async_copy(
                    tab_hbm.at[:, pl.ds(0, BLK)], g_v.at[s], sems.at[s]
                ).wait()
                t = v16[k]
                rl = t & 127
                for q in range(dim // LANES):
                    acc = jnp.zeros((LANES,), jnp.float32)
                    for l in range(LANES):
                        d = q * LANES + l
                        w = g_v[s, d, pl.ds(rl, LANES)]
                        acc = acc + onehot[l] * jnp.full(
                            (LANES,), w[0], jnp.float32)
                    o_v[i & (b_per_w // 2 - 1), pl.ds(q * LANES, LANES)] = acc
                tn = (v16[k + DEPTH] if k + DEPTH < LANES
                      else vn16[k + DEPTH - LANES])

                @pl.when(i + DEPTH < b_per_w)
                def _():
                    fetch(tn, s)

            @pl.when(g == n_grp // 2 - 1)
            def _():
                pltpu.sync_copy(
                    o_v, out_hbm.at[pl.ds(base, b_per_w // 2)])

        pl.loop(0, n_grp)(group_body)
        pltpu.sync_copy(
            o_v, out_hbm.at[pl.ds(base + b_per_w // 2, b_per_w // 2)])

    return gather_kernel


def kernel(y, table):
    batch = y.shape[0]
    dim = table.shape[1]
    idx = y.reshape(batch).astype(jnp.int32)
    out = _build(batch, dim)(idx, table.T)
    return out.reshape(batch, 1, dim)
